# Initial kernel scaffold; baseline (speedup 1.0000x reference)
import jax
import jax.numpy as jnp
from jax.experimental import pallas as pl


def kernel(points, batch, W1, b1, W2, b2, W3, b3, Wg1, bg1, Wg2, bg2, Wg3, bg3):
    raise NotImplementedError("write your pallas kernel here")


# speedup vs baseline: 14.7272x; 14.7272x over previous
import functools

import jax
import jax.numpy as jnp
from jax import lax
from jax.experimental import pallas as pl
from jax.experimental.pallas import tpu as pltpu
from jax.experimental.pallas import tpu_sc as plsc

B, P, S, K = 16, 2048, 256, 32
RADIUS = 0.15
R2 = RADIUS * RADIUS
NEG_INF = float("-inf")


def _fps_body(px_ref, py_ref, pz_ref, sel_ref, fx_ref, fy_ref, fz_ref):
    px = px_ref[...]
    py = py_ref[...]
    pz = pz_ref[...]
    iota_p = lax.broadcasted_iota(jnp.int32, (B, P), 1)
    iota_s = lax.broadcasted_iota(jnp.int32, (B, S), 1)

    lx0 = px[:, 0:1]
    ly0 = py[:, 0:1]
    lz0 = pz[:, 0:1]
    sel0 = jnp.zeros((B, S), jnp.int32)
    fx0 = jnp.where(iota_s == 0, lx0, 0.0)
    fy0 = jnp.where(iota_s == 0, ly0, 0.0)
    fz0 = jnp.where(iota_s == 0, lz0, 0.0)
    dist0 = jnp.full((B, P), jnp.inf, jnp.float32)

    def body(i, carry):
        dist, sel, fx, fy, fz, lx, ly, lz = carry
        dx = px - lx
        dy = py - ly
        dz = pz - lz
        d = (dx * dx + dy * dy) + dz * dz
        dist = jnp.minimum(dist, d)
        m = jnp.max(dist, axis=1, keepdims=True)
        nxt = jnp.min(jnp.where(dist == m, iota_p, P), axis=1, keepdims=True)
        onehot = iota_p == nxt
        lx = jnp.sum(jnp.where(onehot, px, 0.0), axis=1, keepdims=True)
        ly = jnp.sum(jnp.where(onehot, py, 0.0), axis=1, keepdims=True)
        lz = jnp.sum(jnp.where(onehot, pz, 0.0), axis=1, keepdims=True)
        at_i = iota_s == i
        sel = jnp.where(at_i, nxt, sel)
        fx = jnp.where(at_i, lx, fx)
        fy = jnp.where(at_i, ly, fy)
        fz = jnp.where(at_i, lz, fz)
        return (dist, sel, fx, fy, fz, lx, ly, lz)

    carry = (dist0, sel0, fx0, fy0, fz0, lx0, ly0, lz0)
    _, sel, fx, fy, fz, _, _, _ = lax.fori_loop(1, S, body, carry)
    sel_ref[...] = sel
    fx_ref[...] = fx
    fy_ref[...] = fy
    fz_ref[...] = fz


def _run_fps(px, py, pz):
    return pl.pallas_call(
        _fps_body,
        out_shape=(
            jax.ShapeDtypeStruct((B, S), jnp.int32),
            jax.ShapeDtypeStruct((B, S), jnp.float32),
            jax.ShapeDtypeStruct((B, S), jnp.float32),
            jax.ShapeDtypeStruct((B, S), jnp.float32),
        ),
    )(px, py, pz)


NWORK = 32
CPW = (B * S) // NWORK
CAND = 2064


def _sc_body(ptsx, ptsy, ptsz, batch, fx, fy, fz, sel,
             relx_o, rely_o, relz_o, fpsb_o,
             px_v, py_v, pz_v, bat_v, cx_v, cy_v, cz_v, sel_v,
             cd2_v, cix_v, pick_v, rx_v, ry_v, rz_v, fpsb_v):
    wid = lax.axis_index("s") * 2 + lax.axis_index("c")
    b = wid // 2
    half = wid % 2

    pltpu.sync_copy(ptsx.at[b], px_v)
    pltpu.sync_copy(ptsy.at[b], py_v)
    pltpu.sync_copy(ptsz.at[b], pz_v)
    pltpu.sync_copy(batch.at[pl.ds(b * P, P)], bat_v)
    pltpu.sync_copy(fx.at[b, pl.ds(half * CPW, CPW)], cx_v)
    pltpu.sync_copy(fy.at[b, pl.ds(half * CPW, CPW)], cy_v)
    pltpu.sync_copy(fz.at[b, pl.ds(half * CPW, CPW)], cz_v)
    pltpu.sync_copy(sel.at[b, pl.ds(half * CPW, CPW)], sel_v)

    lane = lax.iota(jnp.int32, 16)

    for j in range(CPW // 16):
        sv = sel_v[pl.ds(j * 16, 16)]
        fpsb_v[pl.ds(j * 16, 16)] = plsc.load_gather(bat_v, [sv])

    inf16 = jnp.full((16,), jnp.inf, jnp.float32)

    def center_body(i, _):
        spl_i = jnp.full((16,), 0, jnp.int32) + i
        cxs = plsc.load_gather(cx_v, [spl_i])
        cys = plsc.load_gather(cy_v, [spl_i])
        czs = plsc.load_gather(cz_v, [spl_i])
        ctr_idx = plsc.load_gather(sel_v, [spl_i])

        for j in range(8):
            cd2_v[pl.ds(j * 16, 16)] = inf16
        for j in range(3):
            pick_v[pl.ds(j * 16, 16)] = ctr_idx

        def scan_body(j, cnt):
            off = j * 16
            pxv = px_v[pl.ds(off, 16)]
            pyv = py_v[pl.ds(off, 16)]
            pzv = pz_v[pl.ds(off, 16)]
            dx = pxv - cxs
            dy = pyv - cys
            dz = pzv - czs
            d2 = (dx * dx + dy * dy) + dz * dz
            m = d2 <= R2
            plsc.store_compressed(cd2_v.at[pl.ds(cnt, 16)], d2, mask=m)
            plsc.store_compressed(cix_v.at[pl.ds(cnt, 16)], lane + off, mask=m)
            return cnt + jnp.sum(m.astype(jnp.int32))

        cnt = lax.fori_loop(0, P // 16, scan_body, jnp.int32(0))

        def run_bisect(_):
            def bis_body(_, lohi):
                lo, hi = lohi
                mid = (lo + hi) * jnp.float32(0.5)

                def cb(j, acc):
                    v = cd2_v[pl.ds(j * 16, 16)]
                    return acc + (v <= mid).astype(jnp.int32)

                accv = lax.fori_loop(0, 8, cb, jnp.zeros((16,), jnp.int32))
                c = jnp.sum(accv)
                ge = c >= K
                return (jnp.where(ge, lo, mid), jnp.where(ge, mid, hi))

            _, hi = lax.fori_loop(
                0, 26, bis_body, (jnp.float32(0.0), jnp.float32(R2)))
            return hi

        t = lax.cond(cnt > K, run_bisect,
                     lambda _: jnp.float32(R2), operand=None)

        def fc_body(j, c2):
            v = cd2_v[pl.ds(j * 16, 16)]
            ix = cix_v[pl.ds(j * 16, 16)]
            m = v <= t
            plsc.store_compressed(pick_v.at[pl.ds(c2, 16)], ix, mask=m)
            return c2 + jnp.sum(m.astype(jnp.int32))

        lax.fori_loop(0, 8, fc_body, jnp.int32(0))

        idx0 = pick_v[pl.ds(0, 16)]
        idx1 = pick_v[pl.ds(16, 16)]
        rinv = jnp.float32(RADIUS)
        rx_v[pl.ds(i * K, 16)] = (plsc.load_gather(px_v, [idx0]) - cxs) / rinv
        ry_v[pl.ds(i * K, 16)] = (plsc.load_gather(py_v, [idx0]) - cys) / rinv
        rz_v[pl.ds(i * K, 16)] = (plsc.load_gather(pz_v, [idx0]) - czs) / rinv
        rx_v[pl.ds(i * K + 16, 16)] = (plsc.load_gather(px_v, [idx1]) - cxs) / rinv
        ry_v[pl.ds(i * K + 16, 16)] = (plsc.load_gather(py_v, [idx1]) - cys) / rinv
        rz_v[pl.ds(i * K + 16, 16)] = (plsc.load_gather(pz_v, [idx1]) - czs) / rinv
        return 0

    lax.fori_loop(0, CPW, center_body, 0)

    base = wid * CPW
    pltpu.sync_copy(rx_v, relx_o.at[pl.ds(base * K, CPW * K)])
    pltpu.sync_copy(ry_v, rely_o.at[pl.ds(base * K, CPW * K)])
    pltpu.sync_copy(rz_v, relz_o.at[pl.ds(base * K, CPW * K)])
    pltpu.sync_copy(fpsb_v, fpsb_o.at[pl.ds(base, CPW)])


def _run_sc(ptsx, ptsy, ptsz, batch, fx, fy, fz, sel):
    mesh = plsc.VectorSubcoreMesh(core_axis_name="c", subcore_axis_name="s")
    f32 = jnp.float32
    i32 = jnp.int32
    kfn = pl.kernel(
        _sc_body,
        mesh=mesh,
        compiler_params=pltpu.CompilerParams(needs_layout_passes=False),
        out_type=(
            jax.ShapeDtypeStruct((B * S * K,), f32),
            jax.ShapeDtypeStruct((B * S * K,), f32),
            jax.ShapeDtypeStruct((B * S * K,), f32),
            jax.ShapeDtypeStruct((B * S,), i32),
        ),
        scratch_types=[
            pltpu.VMEM((P,), f32),
            pltpu.VMEM((P,), f32),
            pltpu.VMEM((P,), f32),
            pltpu.VMEM((P,), i32),
            pltpu.VMEM((CPW,), f32),
            pltpu.VMEM((CPW,), f32),
            pltpu.VMEM((CPW,), f32),
            pltpu.VMEM((CPW,), i32),
            pltpu.VMEM((CAND,), f32),
            pltpu.VMEM((CAND,), i32),
            pltpu.VMEM((CAND,), i32),
            pltpu.VMEM((CPW * K,), f32),
            pltpu.VMEM((CPW * K,), f32),
            pltpu.VMEM((CPW * K,), f32),
            pltpu.VMEM((CPW,), i32),
        ],
    )
    return kfn(ptsx, ptsy, ptsz, batch, fx, fy, fz, sel)


CH = 256
GRID = (B * S) // CH


def _mlp_body(rel8_ref,
              w1_ref, b1_ref, w2_ref, b2_ref, w3_ref, b3_ref,
              wg1_ref, bg1_ref, wg2_ref, bg2_ref, wg3_ref, bg3_ref,
              out_ref):
    E = CH * K
    h = jnp.dot(rel8_ref[...], w1_ref[...],
                preferred_element_type=jnp.float32) + b1_ref[...]
    h = jnp.maximum(h, 0.0)
    h = jnp.dot(h, w2_ref[...], preferred_element_type=jnp.float32) + b2_ref[...]
    h = jnp.maximum(h, 0.0)
    h = jnp.dot(h, w3_ref[...], preferred_element_type=jnp.float32) + b3_ref[...]
    h = jnp.maximum(h, 0.0)

    pooled = jnp.max(h.reshape(CH, K, 128), axis=1)

    g = jnp.dot(pooled, wg1_ref[...], preferred_element_type=jnp.float32) + bg1_ref[...]
    g = jnp.maximum(g, 0.0)
    g = jnp.dot(g, wg2_ref[...], preferred_element_type=jnp.float32) + bg2_ref[...]
    g = jnp.maximum(g, 0.0)
    g = jnp.dot(g, wg3_ref[...], preferred_element_type=jnp.float32) + bg3_ref[...]
    g = jnp.maximum(g, 0.0)
    out_ref[...] = g


def _run_mlp(rel8, w1p, b1, w2t, b2, w3t, b3,
             wg1t, bg1, wg2t, bg2, wg3t, bg3):
    N = B * S
    full = lambda shape: pl.BlockSpec(shape, lambda g: (0,) * len(shape))
    return pl.pallas_call(
        _mlp_body,
        grid=(GRID,),
        in_specs=[
            pl.BlockSpec((CH * K, 8), lambda g: (g, 0)),
            full((8, 64)), full((1, 64)),
            full((64, 64)), full((1, 64)),
            full((64, 128)), full((1, 128)),
            full((128, 128)), full((1, 128)),
            full((128, 256)), full((1, 256)),
            full((256, 256)), full((1, 256)),
        ],
        out_specs=pl.BlockSpec((CH, 256), lambda g: (g, 0)),
        out_shape=jax.ShapeDtypeStruct((N, 256), jnp.float32),
    )(rel8, w1p, b1, w2t, b2, w3t, b3,
      wg1t, bg1, wg2t, bg2, wg3t, bg3)


def kernel(points, batch, W1, b1, W2, b2, W3, b3, Wg1, bg1, Wg2, bg2, Wg3, bg3):
    pts = points.reshape(B, P, 3)
    px = pts[:, :, 0]
    py = pts[:, :, 1]
    pz = pts[:, :, 2]

    sel, fx, fy, fz = _run_fps(px, py, pz)

    relx, rely, relz, fpsb = _run_sc(px, py, pz, batch, fx, fy, fz, sel)

    E = B * S * K
    rel8 = jnp.concatenate(
        [relx.reshape(E, 1), rely.reshape(E, 1), relz.reshape(E, 1),
         jnp.zeros((E, 5), jnp.float32)], axis=1)
    w1p = jnp.concatenate([W1.T, jnp.zeros((5, 64), jnp.float32)], axis=0)

    g = _run_mlp(rel8,
                 w1p, b1.reshape(1, -1), W2.T, b2.reshape(1, -1),
                 W3.T, b3.reshape(1, -1), Wg1.T, bg1.reshape(1, -1),
                 Wg2.T, bg2.reshape(1, -1), Wg3.T, bg3.reshape(1, -1))

    fps_pts = jnp.stack([fx, fy, fz], axis=-1).reshape(B * S, 3)
    return fps_pts, g, fpsb.reshape(B * S)


# speedup vs baseline: 14.8907x; 1.0111x over previous
import functools

import jax
import jax.numpy as jnp
from jax import lax
from jax.experimental import pallas as pl
from jax.experimental.pallas import tpu as pltpu
from jax.experimental.pallas import tpu_sc as plsc

B, P, S, K = 16, 2048, 256, 32
RADIUS = 0.15
R2 = RADIUS * RADIUS
NEG_INF = float("-inf")


def _fps_body(px_ref, py_ref, pz_ref, sel_ref, fx_ref, fy_ref, fz_ref):
    px = px_ref[...]
    py = py_ref[...]
    pz = pz_ref[...]
    iota_p = lax.broadcasted_iota(jnp.int32, (B, P), 1)
    iota_s = lax.broadcasted_iota(jnp.int32, (B, S), 1)

    lx0 = px[:, 0:1]
    ly0 = py[:, 0:1]
    lz0 = pz[:, 0:1]
    sel0 = jnp.zeros((B, S), jnp.int32)
    fx0 = jnp.where(iota_s == 0, lx0, 0.0)
    fy0 = jnp.where(iota_s == 0, ly0, 0.0)
    fz0 = jnp.where(iota_s == 0, lz0, 0.0)
    dist0 = jnp.full((B, P), jnp.inf, jnp.float32)

    def body(i, carry):
        dist, sel, fx, fy, fz, lx, ly, lz = carry
        dx = px - lx
        dy = py - ly
        dz = pz - lz
        d = (dx * dx + dy * dy) + dz * dz
        dist = jnp.minimum(dist, d)
        m = jnp.max(dist, axis=1, keepdims=True)
        nxt = jnp.min(jnp.where(dist == m, iota_p, P), axis=1, keepdims=True)
        onehot = iota_p == nxt
        lx = jnp.sum(jnp.where(onehot, px, 0.0), axis=1, keepdims=True)
        ly = jnp.sum(jnp.where(onehot, py, 0.0), axis=1, keepdims=True)
        lz = jnp.sum(jnp.where(onehot, pz, 0.0), axis=1, keepdims=True)
        at_i = iota_s == i
        sel = jnp.where(at_i, nxt, sel)
        fx = jnp.where(at_i, lx, fx)
        fy = jnp.where(at_i, ly, fy)
        fz = jnp.where(at_i, lz, fz)
        return (dist, sel, fx, fy, fz, lx, ly, lz)

    carry = (dist0, sel0, fx0, fy0, fz0, lx0, ly0, lz0)
    _, sel, fx, fy, fz, _, _, _ = lax.fori_loop(1, S, body, carry)
    sel_ref[...] = sel
    fx_ref[...] = fx
    fy_ref[...] = fy
    fz_ref[...] = fz


def _run_fps(px, py, pz):
    return pl.pallas_call(
        _fps_body,
        out_shape=(
            jax.ShapeDtypeStruct((B, S), jnp.int32),
            jax.ShapeDtypeStruct((B, S), jnp.float32),
            jax.ShapeDtypeStruct((B, S), jnp.float32),
            jax.ShapeDtypeStruct((B, S), jnp.float32),
        ),
    )(px, py, pz)


NWORK = 32
CPW = (B * S) // NWORK
CAND = 2064


def _sc_body(ptsx, ptsy, ptsz, batch, fx, fy, fz, sel,
             relx_o, rely_o, relz_o, fpsb_o,
             px_v, py_v, pz_v, bat_v, cx_v, cy_v, cz_v, sel_v,
             cd2_v, cix_v, pick_v, rx_v, ry_v, rz_v, fpsb_v):
    wid = lax.axis_index("s") * 2 + lax.axis_index("c")
    b = wid // 2
    half = wid % 2

    pltpu.sync_copy(ptsx.at[b], px_v)
    pltpu.sync_copy(ptsy.at[b], py_v)
    pltpu.sync_copy(ptsz.at[b], pz_v)
    pltpu.sync_copy(batch.at[pl.ds(b * P, P)], bat_v)
    pltpu.sync_copy(fx.at[b, pl.ds(half * CPW, CPW)], cx_v)
    pltpu.sync_copy(fy.at[b, pl.ds(half * CPW, CPW)], cy_v)
    pltpu.sync_copy(fz.at[b, pl.ds(half * CPW, CPW)], cz_v)
    pltpu.sync_copy(sel.at[b, pl.ds(half * CPW, CPW)], sel_v)

    lane = lax.iota(jnp.int32, 16)

    for j in range(CPW // 16):
        sv = sel_v[pl.ds(j * 16, 16)]
        fpsb_v[pl.ds(j * 16, 16)] = plsc.load_gather(bat_v, [sv])

    inf16 = jnp.full((16,), jnp.inf, jnp.float32)

    def center_body(i, _):
        spl_i = jnp.full((16,), 0, jnp.int32) + i
        cxs = plsc.load_gather(cx_v, [spl_i])
        cys = plsc.load_gather(cy_v, [spl_i])
        czs = plsc.load_gather(cz_v, [spl_i])
        ctr_idx = plsc.load_gather(sel_v, [spl_i])

        for j in range(8):
            cd2_v[pl.ds(j * 16, 16)] = inf16
        for j in range(3):
            pick_v[pl.ds(j * 16, 16)] = ctr_idx

        def scan_body(j, cnt):
            off = j * 16
            pxv = px_v[pl.ds(off, 16)]
            pyv = py_v[pl.ds(off, 16)]
            pzv = pz_v[pl.ds(off, 16)]
            dx = pxv - cxs
            dy = pyv - cys
            dz = pzv - czs
            d2 = (dx * dx + dy * dy) + dz * dz
            m = d2 <= R2
            plsc.store_compressed(cd2_v.at[pl.ds(cnt, 16)], d2, mask=m)
            plsc.store_compressed(cix_v.at[pl.ds(cnt, 16)], lane + off, mask=m)
            return cnt + plsc.all_reduce_population_count(m)[0]

        cnt = lax.fori_loop(0, P // 16, scan_body, jnp.int32(0))

        def run_bisect(_):
            def bis_body(_, lohi):
                lo, hi = lohi
                mid = (lo + hi) * jnp.float32(0.5)

                def cb(j, acc):
                    v = cd2_v[pl.ds(j * 16, 16)]
                    return acc + plsc.all_reduce_population_count(v <= mid)

                accv = lax.fori_loop(0, 8, cb, jnp.zeros((16,), jnp.int32))
                c = accv[0]
                ge = c >= K
                return (jnp.where(ge, lo, mid), jnp.where(ge, mid, hi))

            _, hi = lax.fori_loop(
                0, 26, bis_body, (jnp.float32(0.0), jnp.float32(R2)))
            return hi

        t = lax.cond(cnt > K, run_bisect,
                     lambda _: jnp.float32(R2), operand=None)

        def fc_body(j, c2):
            v = cd2_v[pl.ds(j * 16, 16)]
            ix = cix_v[pl.ds(j * 16, 16)]
            m = v <= t
            plsc.store_compressed(pick_v.at[pl.ds(c2, 16)], ix, mask=m)
            return c2 + plsc.all_reduce_population_count(m)[0]

        lax.fori_loop(0, 8, fc_body, jnp.int32(0))

        idx0 = pick_v[pl.ds(0, 16)]
        idx1 = pick_v[pl.ds(16, 16)]
        rinv = jnp.float32(RADIUS)
        rx_v[pl.ds(i * K, 16)] = (plsc.load_gather(px_v, [idx0]) - cxs) / rinv
        ry_v[pl.ds(i * K, 16)] = (plsc.load_gather(py_v, [idx0]) - cys) / rinv
        rz_v[pl.ds(i * K, 16)] = (plsc.load_gather(pz_v, [idx0]) - czs) / rinv
        rx_v[pl.ds(i * K + 16, 16)] = (plsc.load_gather(px_v, [idx1]) - cxs) / rinv
        ry_v[pl.ds(i * K + 16, 16)] = (plsc.load_gather(py_v, [idx1]) - cys) / rinv
        rz_v[pl.ds(i * K + 16, 16)] = (plsc.load_gather(pz_v, [idx1]) - czs) / rinv
        return 0

    lax.fori_loop(0, CPW, center_body, 0)

    base = wid * CPW
    pltpu.sync_copy(rx_v, relx_o.at[pl.ds(base * K, CPW * K)])
    pltpu.sync_copy(ry_v, rely_o.at[pl.ds(base * K, CPW * K)])
    pltpu.sync_copy(rz_v, relz_o.at[pl.ds(base * K, CPW * K)])
    pltpu.sync_copy(fpsb_v, fpsb_o.at[pl.ds(base, CPW)])


def _run_sc(ptsx, ptsy, ptsz, batch, fx, fy, fz, sel):
    mesh = plsc.VectorSubcoreMesh(core_axis_name="c", subcore_axis_name="s")
    f32 = jnp.float32
    i32 = jnp.int32
    kfn = pl.kernel(
        _sc_body,
        mesh=mesh,
        compiler_params=pltpu.CompilerParams(needs_layout_passes=False),
        out_type=(
            jax.ShapeDtypeStruct((B * S * K,), f32),
            jax.ShapeDtypeStruct((B * S * K,), f32),
            jax.ShapeDtypeStruct((B * S * K,), f32),
            jax.ShapeDtypeStruct((B * S,), i32),
        ),
        scratch_types=[
            pltpu.VMEM((P,), f32),
            pltpu.VMEM((P,), f32),
            pltpu.VMEM((P,), f32),
            pltpu.VMEM((P,), i32),
            pltpu.VMEM((CPW,), f32),
            pltpu.VMEM((CPW,), f32),
            pltpu.VMEM((CPW,), f32),
            pltpu.VMEM((CPW,), i32),
            pltpu.VMEM((CAND,), f32),
            pltpu.VMEM((CAND,), i32),
            pltpu.VMEM((CAND,), i32),
            pltpu.VMEM((CPW * K,), f32),
            pltpu.VMEM((CPW * K,), f32),
            pltpu.VMEM((CPW * K,), f32),
            pltpu.VMEM((CPW,), i32),
        ],
    )
    return kfn(ptsx, ptsy, ptsz, batch, fx, fy, fz, sel)


CH = 256
GRID = (B * S) // CH


def _mlp_body(rel8_ref,
              w1_ref, b1_ref, w2_ref, b2_ref, w3_ref, b3_ref,
              wg1_ref, bg1_ref, wg2_ref, bg2_ref, wg3_ref, bg3_ref,
              out_ref):
    E = CH * K
    h = jnp.dot(rel8_ref[...], w1_ref[...],
                preferred_element_type=jnp.float32) + b1_ref[...]
    h = jnp.maximum(h, 0.0)
    h = jnp.dot(h, w2_ref[...], preferred_element_type=jnp.float32) + b2_ref[...]
    h = jnp.maximum(h, 0.0)
    h = jnp.dot(h, w3_ref[...], preferred_element_type=jnp.float32) + b3_ref[...]
    h = jnp.maximum(h, 0.0)

    pooled = jnp.max(h.reshape(CH, K, 128), axis=1)

    g = jnp.dot(pooled, wg1_ref[...], preferred_element_type=jnp.float32) + bg1_ref[...]
    g = jnp.maximum(g, 0.0)
    g = jnp.dot(g, wg2_ref[...], preferred_element_type=jnp.float32) + bg2_ref[...]
    g = jnp.maximum(g, 0.0)
    g = jnp.dot(g, wg3_ref[...], preferred_element_type=jnp.float32) + bg3_ref[...]
    g = jnp.maximum(g, 0.0)
    out_ref[...] = g


def _run_mlp(rel8, w1p, b1, w2t, b2, w3t, b3,
             wg1t, bg1, wg2t, bg2, wg3t, bg3):
    N = B * S
    full = lambda shape: pl.BlockSpec(shape, lambda g: (0,) * len(shape))
    return pl.pallas_call(
        _mlp_body,
        grid=(GRID,),
        in_specs=[
            pl.BlockSpec((CH * K, 8), lambda g: (g, 0)),
            full((8, 64)), full((1, 64)),
            full((64, 64)), full((1, 64)),
            full((64, 128)), full((1, 128)),
            full((128, 128)), full((1, 128)),
            full((128, 256)), full((1, 256)),
            full((256, 256)), full((1, 256)),
        ],
        out_specs=pl.BlockSpec((CH, 256), lambda g: (g, 0)),
        out_shape=jax.ShapeDtypeStruct((N, 256), jnp.float32),
    )(rel8, w1p, b1, w2t, b2, w3t, b3,
      wg1t, bg1, wg2t, bg2, wg3t, bg3)


def kernel(points, batch, W1, b1, W2, b2, W3, b3, Wg1, bg1, Wg2, bg2, Wg3, bg3):
    pts = points.reshape(B, P, 3)
    px = pts[:, :, 0]
    py = pts[:, :, 1]
    pz = pts[:, :, 2]

    sel, fx, fy, fz = _run_fps(px, py, pz)

    relx, rely, relz, fpsb = _run_sc(px, py, pz, batch, fx, fy, fz, sel)

    E = B * S * K
    rel8 = jnp.concatenate(
        [relx.reshape(E, 1), rely.reshape(E, 1), relz.reshape(E, 1),
         jnp.zeros((E, 5), jnp.float32)], axis=1)
    w1p = jnp.concatenate([W1.T, jnp.zeros((5, 64), jnp.float32)], axis=0)

    g = _run_mlp(rel8,
                 w1p, b1.reshape(1, -1), W2.T, b2.reshape(1, -1),
                 W3.T, b3.reshape(1, -1), Wg1.T, bg1.reshape(1, -1),
                 Wg2.T, bg2.reshape(1, -1), Wg3.T, bg3.reshape(1, -1))

    fps_pts = jnp.stack([fx, fy, fz], axis=-1).reshape(B * S, 3)
    return fps_pts, g, fpsb.reshape(B * S)


# speedup vs baseline: 22.9540x; 1.5415x over previous
import functools

import jax
import jax.numpy as jnp
from jax import lax
from jax.experimental import pallas as pl
from jax.experimental.pallas import tpu as pltpu
from jax.experimental.pallas import tpu_sc as plsc

B, P, S, K = 16, 2048, 256, 32
RADIUS = 0.15
R2 = RADIUS * RADIUS
NEG_INF = float("-inf")


def _fps_body(px_ref, py_ref, pz_ref, sel_ref, fx_ref, fy_ref, fz_ref):
    px = px_ref[...]
    py = py_ref[...]
    pz = pz_ref[...]
    iota_p = lax.broadcasted_iota(jnp.int32, (B, P), 1)
    iota_s = lax.broadcasted_iota(jnp.int32, (B, S), 1)

    lx0 = px[:, 0:1]
    ly0 = py[:, 0:1]
    lz0 = pz[:, 0:1]
    sel0 = jnp.zeros((B, S), jnp.int32)
    fx0 = jnp.where(iota_s == 0, lx0, 0.0)
    fy0 = jnp.where(iota_s == 0, ly0, 0.0)
    fz0 = jnp.where(iota_s == 0, lz0, 0.0)
    dist0 = jnp.full((B, P), jnp.inf, jnp.float32)

    def body(i, carry):
        dist, sel, fx, fy, fz, lx, ly, lz = carry
        dx = px - lx
        dy = py - ly
        dz = pz - lz
        d = (dx * dx + dy * dy) + dz * dz
        dist = jnp.minimum(dist, d)
        m = jnp.max(dist, axis=1, keepdims=True)
        nxt = jnp.min(jnp.where(dist == m, iota_p, P), axis=1, keepdims=True)
        onehot = iota_p == nxt
        lx = jnp.sum(jnp.where(onehot, px, 0.0), axis=1, keepdims=True)
        ly = jnp.sum(jnp.where(onehot, py, 0.0), axis=1, keepdims=True)
        lz = jnp.sum(jnp.where(onehot, pz, 0.0), axis=1, keepdims=True)
        at_i = iota_s == i
        sel = jnp.where(at_i, nxt, sel)
        fx = jnp.where(at_i, lx, fx)
        fy = jnp.where(at_i, ly, fy)
        fz = jnp.where(at_i, lz, fz)
        return (dist, sel, fx, fy, fz, lx, ly, lz)

    carry = (dist0, sel0, fx0, fy0, fz0, lx0, ly0, lz0)
    _, sel, fx, fy, fz, _, _, _ = lax.fori_loop(1, S, body, carry)
    sel_ref[...] = sel
    fx_ref[...] = fx
    fy_ref[...] = fy
    fz_ref[...] = fz


def _run_fps(px, py, pz):
    return pl.pallas_call(
        _fps_body,
        out_shape=(
            jax.ShapeDtypeStruct((B, S), jnp.int32),
            jax.ShapeDtypeStruct((B, S), jnp.float32),
            jax.ShapeDtypeStruct((B, S), jnp.float32),
            jax.ShapeDtypeStruct((B, S), jnp.float32),
        ),
    )(px, py, pz)


NWORK = 32
CPW = (B * S) // NWORK
CAND = 2064


def _sc_body(ptsx, ptsy, ptsz, batch, fx, fy, fz, sel,
             relx_o, rely_o, relz_o, fpsb_o,
             px_v, py_v, pz_v, bat_v, cx_v, cy_v, cz_v, sel_v,
             cd2_v, cix_v, pick_v, rx_v, ry_v, rz_v, fpsb_v):
    wid = lax.axis_index("s") * 2 + lax.axis_index("c")
    b = wid // 2
    half = wid % 2

    pltpu.sync_copy(ptsx.at[b], px_v)
    pltpu.sync_copy(ptsy.at[b], py_v)
    pltpu.sync_copy(ptsz.at[b], pz_v)
    pltpu.sync_copy(batch.at[pl.ds(b * P, P)], bat_v)
    pltpu.sync_copy(fx.at[b, pl.ds(half * CPW, CPW)], cx_v)
    pltpu.sync_copy(fy.at[b, pl.ds(half * CPW, CPW)], cy_v)
    pltpu.sync_copy(fz.at[b, pl.ds(half * CPW, CPW)], cz_v)
    pltpu.sync_copy(sel.at[b, pl.ds(half * CPW, CPW)], sel_v)

    lane = lax.iota(jnp.int32, 16)

    for j in range(CPW // 16):
        sv = sel_v[pl.ds(j * 16, 16)]
        fpsb_v[pl.ds(j * 16, 16)] = plsc.load_gather(bat_v, [sv])

    inf16 = jnp.full((16,), jnp.inf, jnp.float32)

    def center_body(i, _):
        spl_i = jnp.full((16,), 0, jnp.int32) + i
        cxs = plsc.load_gather(cx_v, [spl_i])
        cys = plsc.load_gather(cy_v, [spl_i])
        czs = plsc.load_gather(cz_v, [spl_i])
        ctr_idx = plsc.load_gather(sel_v, [spl_i])

        for j in range(8):
            cd2_v[pl.ds(j * 16, 16)] = inf16
        for j in range(3):
            pick_v[pl.ds(j * 16, 16)] = ctr_idx

        @plsc.parallel_loop(0, P // 16, unroll=4,
                            carry=jnp.zeros((16,), jnp.int32))
        def scan_body(j, cnt_vec):
            off = j * 16
            pxv = px_v[pl.ds(off, 16)]
            pyv = py_v[pl.ds(off, 16)]
            pzv = pz_v[pl.ds(off, 16)]
            dx = pxv - cxs
            dy = pyv - cys
            dz = pzv - czs
            d2 = (dx * dx + dy * dy) + dz * dz
            m = d2 <= R2
            mi = m.astype(jnp.int32)
            tgt = cnt_vec + plsc.cumsum(mi) - mi
            plsc.store_scatter(cd2_v, [tgt], d2, mask=m)
            plsc.store_scatter(cix_v, [tgt], lane + off, mask=m)
            return cnt_vec + plsc.all_reduce_population_count(m)

        cnt = scan_body[0]

        def run_bisect(_):
            def bis_body(_, lohi):
                lo, hi = lohi
                mid = (lo + hi) * 0.5
                acc = jnp.zeros((16,), jnp.int32)
                for j in range(8):
                    v = cd2_v[pl.ds(j * 16, 16)]
                    acc = acc + plsc.all_reduce_population_count(v <= mid)
                ge = acc >= K
                return (jnp.where(ge, lo, mid), jnp.where(ge, mid, hi))

            lohi0 = (jnp.zeros((16,), jnp.float32),
                     jnp.full((16,), R2, jnp.float32))
            _, hi = lax.fori_loop(0, 26, bis_body, lohi0)
            return hi

        t = lax.cond(cnt > K, run_bisect,
                     lambda _: jnp.full((16,), R2, jnp.float32), operand=None)

        c2_vec = jnp.zeros((16,), jnp.int32)
        for j in range(8):
            v = cd2_v[pl.ds(j * 16, 16)]
            ix = cix_v[pl.ds(j * 16, 16)]
            m = v <= t
            mi = m.astype(jnp.int32)
            tgt = c2_vec + plsc.cumsum(mi) - mi
            plsc.store_scatter(pick_v, [tgt], ix, mask=m)
            c2_vec = c2_vec + plsc.all_reduce_population_count(m)

        idx0 = pick_v[pl.ds(0, 16)]
        idx1 = pick_v[pl.ds(16, 16)]
        rinv = jnp.float32(RADIUS)
        rx_v[pl.ds(i * K, 16)] = (plsc.load_gather(px_v, [idx0]) - cxs) / rinv
        ry_v[pl.ds(i * K, 16)] = (plsc.load_gather(py_v, [idx0]) - cys) / rinv
        rz_v[pl.ds(i * K, 16)] = (plsc.load_gather(pz_v, [idx0]) - czs) / rinv
        rx_v[pl.ds(i * K + 16, 16)] = (plsc.load_gather(px_v, [idx1]) - cxs) / rinv
        ry_v[pl.ds(i * K + 16, 16)] = (plsc.load_gather(py_v, [idx1]) - cys) / rinv
        rz_v[pl.ds(i * K + 16, 16)] = (plsc.load_gather(pz_v, [idx1]) - czs) / rinv
        return 0

    lax.fori_loop(0, CPW, center_body, 0)

    base = wid * CPW
    pltpu.sync_copy(rx_v, relx_o.at[pl.ds(base * K, CPW * K)])
    pltpu.sync_copy(ry_v, rely_o.at[pl.ds(base * K, CPW * K)])
    pltpu.sync_copy(rz_v, relz_o.at[pl.ds(base * K, CPW * K)])
    pltpu.sync_copy(fpsb_v, fpsb_o.at[pl.ds(base, CPW)])


def _run_sc(ptsx, ptsy, ptsz, batch, fx, fy, fz, sel):
    mesh = plsc.VectorSubcoreMesh(core_axis_name="c", subcore_axis_name="s")
    f32 = jnp.float32
    i32 = jnp.int32
    kfn = pl.kernel(
        _sc_body,
        mesh=mesh,
        compiler_params=pltpu.CompilerParams(needs_layout_passes=False),
        out_type=(
            jax.ShapeDtypeStruct((B * S * K,), f32),
            jax.ShapeDtypeStruct((B * S * K,), f32),
            jax.ShapeDtypeStruct((B * S * K,), f32),
            jax.ShapeDtypeStruct((B * S,), i32),
        ),
        scratch_types=[
            pltpu.VMEM((P,), f32),
            pltpu.VMEM((P,), f32),
            pltpu.VMEM((P,), f32),
            pltpu.VMEM((P,), i32),
            pltpu.VMEM((CPW,), f32),
            pltpu.VMEM((CPW,), f32),
            pltpu.VMEM((CPW,), f32),
            pltpu.VMEM((CPW,), i32),
            pltpu.VMEM((CAND,), f32),
            pltpu.VMEM((CAND,), i32),
            pltpu.VMEM((CAND,), i32),
            pltpu.VMEM((CPW * K,), f32),
            pltpu.VMEM((CPW * K,), f32),
            pltpu.VMEM((CPW * K,), f32),
            pltpu.VMEM((CPW,), i32),
        ],
    )
    return kfn(ptsx, ptsy, ptsz, batch, fx, fy, fz, sel)


CH = 256
GRID = (B * S) // CH


def _mlp_body(rel8_ref,
              w1_ref, b1_ref, w2_ref, b2_ref, w3_ref, b3_ref,
              wg1_ref, bg1_ref, wg2_ref, bg2_ref, wg3_ref, bg3_ref,
              out_ref):
    E = CH * K
    h = jnp.dot(rel8_ref[...], w1_ref[...],
                preferred_element_type=jnp.float32) + b1_ref[...]
    h = jnp.maximum(h, 0.0)
    h = jnp.dot(h, w2_ref[...], preferred_element_type=jnp.float32) + b2_ref[...]
    h = jnp.maximum(h, 0.0)
    h = jnp.dot(h, w3_ref[...], preferred_element_type=jnp.float32) + b3_ref[...]
    h = jnp.maximum(h, 0.0)

    pooled = jnp.max(h.reshape(CH, K, 128), axis=1)

    g = jnp.dot(pooled, wg1_ref[...], preferred_element_type=jnp.float32) + bg1_ref[...]
    g = jnp.maximum(g, 0.0)
    g = jnp.dot(g, wg2_ref[...], preferred_element_type=jnp.float32) + bg2_ref[...]
    g = jnp.maximum(g, 0.0)
    g = jnp.dot(g, wg3_ref[...], preferred_element_type=jnp.float32) + bg3_ref[...]
    g = jnp.maximum(g, 0.0)
    out_ref[...] = g


def _run_mlp(rel8, w1p, b1, w2t, b2, w3t, b3,
             wg1t, bg1, wg2t, bg2, wg3t, bg3):
    N = B * S
    full = lambda shape: pl.BlockSpec(shape, lambda g: (0,) * len(shape))
    return pl.pallas_call(
        _mlp_body,
        grid=(GRID,),
        in_specs=[
            pl.BlockSpec((CH * K, 8), lambda g: (g, 0)),
            full((8, 64)), full((1, 64)),
            full((64, 64)), full((1, 64)),
            full((64, 128)), full((1, 128)),
            full((128, 128)), full((1, 128)),
            full((128, 256)), full((1, 256)),
            full((256, 256)), full((1, 256)),
        ],
        out_specs=pl.BlockSpec((CH, 256), lambda g: (g, 0)),
        out_shape=jax.ShapeDtypeStruct((N, 256), jnp.float32),
    )(rel8, w1p, b1, w2t, b2, w3t, b3,
      wg1t, bg1, wg2t, bg2, wg3t, bg3)


def kernel(points, batch, W1, b1, W2, b2, W3, b3, Wg1, bg1, Wg2, bg2, Wg3, bg3):
    pts = points.reshape(B, P, 3)
    px = pts[:, :, 0]
    py = pts[:, :, 1]
    pz = pts[:, :, 2]

    sel, fx, fy, fz = _run_fps(px, py, pz)

    relx, rely, relz, fpsb = _run_sc(px, py, pz, batch, fx, fy, fz, sel)

    E = B * S * K
    rel8 = jnp.concatenate(
        [relx.reshape(E, 1), rely.reshape(E, 1), relz.reshape(E, 1),
         jnp.zeros((E, 5), jnp.float32)], axis=1)
    w1p = jnp.concatenate([W1.T, jnp.zeros((5, 64), jnp.float32)], axis=0)

    g = _run_mlp(rel8,
                 w1p, b1.reshape(1, -1), W2.T, b2.reshape(1, -1),
                 W3.T, b3.reshape(1, -1), Wg1.T, bg1.reshape(1, -1),
                 Wg2.T, bg2.reshape(1, -1), Wg3.T, bg3.reshape(1, -1))

    fps_pts = jnp.stack([fx, fy, fz], axis=-1).reshape(B * S, 3)
    return fps_pts, g, fpsb.reshape(B * S)
